# Initial kernel scaffold; baseline (speedup 1.0000x reference)
import jax
import jax.numpy as jnp
from jax.experimental import pallas as pl


def kernel(x, W1, W2):
    raise NotImplementedError("write your pallas kernel here")


# speedup vs baseline: 4.6043x; 4.6043x over previous
import functools

import jax
import jax.numpy as jnp
from jax.experimental import pallas as pl
from jax.experimental.pallas import tpu as pltpu

N_TOK = 32768
D_MODEL = 1024
N_EXPERTS = 64
NUM_SELECTS = 8
BLOCK_T = 1024


def _gate_body(x_ref, w1_ref, w2_ref, idx_ref, scr_ref, loss_ref,
               imp_ref, load_ref):
    i = pl.program_id(0)
    nb = pl.num_programs(0)

    h = jnp.tanh(jnp.dot(x_ref[...], w1_ref[...],
                         preferred_element_type=jnp.float32))
    logits = jnp.dot(h, w2_ref[...], preferred_element_type=jnp.float32)
    bt, ne = logits.shape
    iota_e = jax.lax.broadcasted_iota(jnp.int32, (bt, ne), 1)

    l = logits
    idx_cols, val_cols, onehots = [], [], []
    for _ in range(NUM_SELECTS):
        m = jnp.max(l, axis=1, keepdims=True)
        cand = jnp.where(l == m, iota_e, ne)
        idx = jnp.min(cand, axis=1, keepdims=True)
        onehot = iota_e == idx
        idx_cols.append(idx)
        val_cols.append(m)
        onehots.append(onehot)
        l = jnp.where(onehot, -jnp.inf, l)

    vals = jnp.concatenate(val_cols, axis=1)
    idxs = jnp.concatenate(idx_cols, axis=1)
    e = jnp.exp(vals - vals[:, 0:1])
    s = e / jnp.sum(e, axis=1, keepdims=True)

    idx_ref[...] = idxs
    scr_ref[...] = s

    imp = jnp.zeros((bt, ne), jnp.float32)
    ld = jnp.zeros((bt, ne), jnp.float32)
    for k in range(NUM_SELECTS):
        sk = s[:, k:k + 1]
        imp = imp + jnp.where(onehots[k], sk, 0.0)
        ld = ld + jnp.where(onehots[k] & (sk > 0), 1.0, 0.0)
    imp_p = jnp.sum(imp, axis=0, keepdims=True)
    ld_p = jnp.sum(ld, axis=0, keepdims=True)

    @pl.when(i == 0)
    def _init():
        imp_ref[...] = jnp.zeros_like(imp_ref)
        load_ref[...] = jnp.zeros_like(load_ref)

    imp_ref[...] += imp_p
    load_ref[...] += ld_p

    @pl.when(i == nb - 1)
    def _fin():
        def cv2(v):
            mean = jnp.sum(v, axis=1, keepdims=True) / ne
            var = jnp.sum((v - mean) ** 2, axis=1, keepdims=True) / (ne - 1)
            return var / (mean ** 2 + 1e-10)

        loss_ref[...] = (cv2(imp_ref[...]) + cv2(load_ref[...])) * 0.01


@jax.jit
def kernel(x, W1, W2):
    nb = N_TOK // BLOCK_T
    idxs, scores, loss = pl.pallas_call(
        _gate_body,
        grid=(nb,),
        in_specs=[
            pl.BlockSpec((BLOCK_T, D_MODEL), lambda i: (i, 0)),
            pl.BlockSpec((D_MODEL, N_EXPERTS), lambda i: (0, 0)),
            pl.BlockSpec((N_EXPERTS, N_EXPERTS), lambda i: (0, 0)),
        ],
        out_specs=[
            pl.BlockSpec((BLOCK_T, NUM_SELECTS), lambda i: (i, 0)),
            pl.BlockSpec((BLOCK_T, NUM_SELECTS), lambda i: (i, 0)),
            pl.BlockSpec((1, 1), lambda i: (0, 0)),
        ],
        out_shape=[
            jax.ShapeDtypeStruct((N_TOK, NUM_SELECTS), jnp.int32),
            jax.ShapeDtypeStruct((N_TOK, NUM_SELECTS), jnp.float32),
            jax.ShapeDtypeStruct((1, 1), jnp.float32),
        ],
        scratch_shapes=[
            pltpu.VMEM((1, N_EXPERTS), jnp.float32),
            pltpu.VMEM((1, N_EXPERTS), jnp.float32),
        ],
        compiler_params=pltpu.CompilerParams(
            dimension_semantics=("arbitrary",),
        ),
    )(x, W1.T, W2.T)
    return idxs, scores, jnp.reshape(loss, ())


# speedup vs baseline: 16.1256x; 3.5023x over previous
import functools

import jax
import jax.numpy as jnp
from jax.experimental import pallas as pl
from jax.experimental.pallas import tpu as pltpu

N_TOK = 32768
D_MODEL = 1024
N_EXPERTS = 64
NUM_SELECTS = 8
BLOCK_T = 1024


def _gate_body(x_ref, w1_ref, w2_ref, idx_ref, scr_ref, loss_ref,
               imp_ref, load_ref):
    i = pl.program_id(0)
    nb = pl.num_programs(0)

    h = jnp.tanh(jnp.dot(x_ref[...], w1_ref[...],
                         preferred_element_type=jnp.float32))
    lt = jax.lax.dot_general(w2_ref[...], h, (((1,), (1,)), ((), ())),
                             preferred_element_type=jnp.float32)
    ne, bt = lt.shape
    iota_e = jax.lax.broadcasted_iota(jnp.int32, (ne, bt), 0)

    l = lt
    idx_rows, val_rows, onehots = [], [], []
    for _ in range(NUM_SELECTS):
        m = jnp.max(l, axis=0, keepdims=True)
        cand = jnp.where(l == m, iota_e, ne)
        idx = jnp.min(cand, axis=0, keepdims=True)
        onehot = iota_e == idx
        idx_rows.append(idx)
        val_rows.append(m)
        onehots.append(onehot)
        l = jnp.where(onehot, -jnp.inf, l)

    vals = jnp.concatenate(val_rows, axis=0)
    idxs = jnp.concatenate(idx_rows, axis=0)
    e = jnp.exp(vals - vals[0:1, :])
    s = e / jnp.sum(e, axis=0, keepdims=True)

    idx_ref[...] = idxs
    scr_ref[...] = s

    imp = jnp.zeros((ne, bt), jnp.float32)
    ld = jnp.zeros((ne, bt), jnp.float32)
    for k in range(NUM_SELECTS):
        sk = s[k:k + 1, :]
        imp = imp + jnp.where(onehots[k], sk, 0.0)
        ld = ld + jnp.where(onehots[k] & (sk > 0), 1.0, 0.0)
    imp_p = jnp.sum(imp, axis=1, keepdims=True)
    ld_p = jnp.sum(ld, axis=1, keepdims=True)

    @pl.when(i == 0)
    def _init():
        imp_ref[...] = jnp.zeros_like(imp_ref)
        load_ref[...] = jnp.zeros_like(load_ref)

    imp_ref[...] += imp_p
    load_ref[...] += ld_p

    @pl.when(i == nb - 1)
    def _fin():
        def cv2(v):
            mean = jnp.sum(v, axis=0, keepdims=True) / ne
            var = jnp.sum((v - mean) ** 2, axis=0, keepdims=True) / (ne - 1)
            return var / (mean ** 2 + 1e-10)

        loss_ref[...] = (cv2(imp_ref[...]) + cv2(load_ref[...])) * 0.01


@jax.jit
def kernel(x, W1, W2):
    nb = N_TOK // BLOCK_T
    idxs_t, scores_t, loss = pl.pallas_call(
        _gate_body,
        grid=(nb,),
        in_specs=[
            pl.BlockSpec((BLOCK_T, D_MODEL), lambda i: (i, 0)),
            pl.BlockSpec((D_MODEL, N_EXPERTS), lambda i: (0, 0)),
            pl.BlockSpec((N_EXPERTS, N_EXPERTS), lambda i: (0, 0)),
        ],
        out_specs=[
            pl.BlockSpec((NUM_SELECTS, BLOCK_T), lambda i: (0, i)),
            pl.BlockSpec((NUM_SELECTS, BLOCK_T), lambda i: (0, i)),
            pl.BlockSpec((1, 1), lambda i: (0, 0)),
        ],
        out_shape=[
            jax.ShapeDtypeStruct((NUM_SELECTS, N_TOK), jnp.int32),
            jax.ShapeDtypeStruct((NUM_SELECTS, N_TOK), jnp.float32),
            jax.ShapeDtypeStruct((1, 1), jnp.float32),
        ],
        scratch_shapes=[
            pltpu.VMEM((N_EXPERTS, 1), jnp.float32),
            pltpu.VMEM((N_EXPERTS, 1), jnp.float32),
        ],
        compiler_params=pltpu.CompilerParams(
            dimension_semantics=("arbitrary",),
        ),
    )(x, W1.T, W2)
    return idxs_t.T, scores_t.T, jnp.reshape(loss, ())


# speedup vs baseline: 17.8789x; 1.1087x over previous
import functools

import jax
import jax.numpy as jnp
from jax.experimental import pallas as pl
from jax.experimental.pallas import tpu as pltpu

N_TOK = 32768
D_MODEL = 1024
N_EXPERTS = 64
NUM_SELECTS = 8
BLOCK_T = 2048


def _gate_body(x_ref, w1_ref, w2_ref, idx_ref, scr_ref, loss_ref,
               imp_ref, load_ref):
    i = pl.program_id(0)
    nb = pl.num_programs(0)

    h = jnp.tanh(jnp.dot(x_ref[...], w1_ref[...],
                         preferred_element_type=jnp.float32))
    lt = jax.lax.dot_general(w2_ref[...], h, (((1,), (1,)), ((), ())),
                             preferred_element_type=jnp.float32)
    ne, bt = lt.shape
    iota_e = jax.lax.broadcasted_iota(jnp.int32, (ne, bt), 0)

    l = lt
    idx_rows, val_rows, onehots = [], [], []
    for _ in range(NUM_SELECTS):
        m = jnp.max(l, axis=0, keepdims=True)
        cand = jnp.where(l == m, iota_e, ne)
        idx = jnp.min(cand, axis=0, keepdims=True)
        onehot = iota_e == idx
        idx_rows.append(idx)
        val_rows.append(m)
        onehots.append(onehot)
        l = jnp.where(onehot, -jnp.inf, l)

    vals = jnp.concatenate(val_rows, axis=0)
    idxs = jnp.concatenate(idx_rows, axis=0)
    e = jnp.exp(vals - vals[0:1, :])
    s = e / jnp.sum(e, axis=0, keepdims=True)

    idx_ref[...] = idxs
    scr_ref[...] = s

    imp = jnp.zeros((ne, bt), jnp.float32)
    ld = jnp.zeros((ne, bt), jnp.float32)
    for k in range(NUM_SELECTS):
        sk = s[k:k + 1, :]
        imp = imp + jnp.where(onehots[k], sk, 0.0)
        ld = ld + jnp.where(onehots[k] & (sk > 0), 1.0, 0.0)
    imp_p = jnp.sum(imp, axis=1, keepdims=True)
    ld_p = jnp.sum(ld, axis=1, keepdims=True)

    @pl.when(i == 0)
    def _init():
        imp_ref[...] = jnp.zeros_like(imp_ref)
        load_ref[...] = jnp.zeros_like(load_ref)

    imp_ref[...] += imp_p
    load_ref[...] += ld_p

    @pl.when(i == nb - 1)
    def _fin():
        def cv2(v):
            mean = jnp.sum(v, axis=0, keepdims=True) / ne
            var = jnp.sum((v - mean) ** 2, axis=0, keepdims=True) / (ne - 1)
            return var / (mean ** 2 + 1e-10)

        loss_ref[...] = (cv2(imp_ref[...]) + cv2(load_ref[...])) * 0.01


@jax.jit
def kernel(x, W1, W2):
    nb = N_TOK // BLOCK_T
    idxs_t, scores_t, loss = pl.pallas_call(
        _gate_body,
        grid=(nb,),
        in_specs=[
            pl.BlockSpec((BLOCK_T, D_MODEL), lambda i: (i, 0)),
            pl.BlockSpec((D_MODEL, N_EXPERTS), lambda i: (0, 0)),
            pl.BlockSpec((N_EXPERTS, N_EXPERTS), lambda i: (0, 0)),
        ],
        out_specs=[
            pl.BlockSpec((NUM_SELECTS, BLOCK_T), lambda i: (0, i)),
            pl.BlockSpec((NUM_SELECTS, BLOCK_T), lambda i: (0, i)),
            pl.BlockSpec((1, 1), lambda i: (0, 0)),
        ],
        out_shape=[
            jax.ShapeDtypeStruct((NUM_SELECTS, N_TOK), jnp.int32),
            jax.ShapeDtypeStruct((NUM_SELECTS, N_TOK), jnp.float32),
            jax.ShapeDtypeStruct((1, 1), jnp.float32),
        ],
        scratch_shapes=[
            pltpu.VMEM((N_EXPERTS, 1), jnp.float32),
            pltpu.VMEM((N_EXPERTS, 1), jnp.float32),
        ],
        compiler_params=pltpu.CompilerParams(
            dimension_semantics=("arbitrary",),
        ),
    )(x, W1.T, W2)
    return idxs_t.T, scores_t.T, jnp.reshape(loss, ())


# speedup vs baseline: 18.1191x; 1.0134x over previous
import functools

import jax
import jax.numpy as jnp
from jax.experimental import pallas as pl
from jax.experimental.pallas import tpu as pltpu

N_TOK = 32768
D_MODEL = 1024
N_EXPERTS = 64
NUM_SELECTS = 8
BLOCK_T = 4096


def _gate_body(x_ref, w1_ref, w2_ref, idx_ref, scr_ref, loss_ref,
               imp_ref, load_ref):
    i = pl.program_id(0)
    nb = pl.num_programs(0)

    h = jnp.tanh(jnp.dot(x_ref[...], w1_ref[...],
                         preferred_element_type=jnp.float32))
    lt = jax.lax.dot_general(w2_ref[...], h, (((1,), (1,)), ((), ())),
                             preferred_element_type=jnp.float32)
    ne, bt = lt.shape
    iota_e = jax.lax.broadcasted_iota(jnp.int32, (ne, bt), 0)

    l = lt
    idx_rows, val_rows, onehots = [], [], []
    for _ in range(NUM_SELECTS):
        m = jnp.max(l, axis=0, keepdims=True)
        cand = jnp.where(l == m, iota_e, ne)
        idx = jnp.min(cand, axis=0, keepdims=True)
        onehot = iota_e == idx
        idx_rows.append(idx)
        val_rows.append(m)
        onehots.append(onehot)
        l = jnp.where(onehot, -jnp.inf, l)

    vals = jnp.concatenate(val_rows, axis=0)
    idxs = jnp.concatenate(idx_rows, axis=0)
    e = jnp.exp(vals - vals[0:1, :])
    s = e / jnp.sum(e, axis=0, keepdims=True)

    idx_ref[...] = idxs
    scr_ref[...] = s

    imp = jnp.zeros((ne, bt), jnp.float32)
    ld = jnp.zeros((ne, bt), jnp.float32)
    for k in range(NUM_SELECTS):
        sk = s[k:k + 1, :]
        imp = imp + jnp.where(onehots[k], sk, 0.0)
        ld = ld + jnp.where(onehots[k] & (sk > 0), 1.0, 0.0)
    imp_p = jnp.sum(imp, axis=1, keepdims=True)
    ld_p = jnp.sum(ld, axis=1, keepdims=True)

    @pl.when(i == 0)
    def _init():
        imp_ref[...] = jnp.zeros_like(imp_ref)
        load_ref[...] = jnp.zeros_like(load_ref)

    imp_ref[...] += imp_p
    load_ref[...] += ld_p

    @pl.when(i == nb - 1)
    def _fin():
        def cv2(v):
            mean = jnp.sum(v, axis=0, keepdims=True) / ne
            var = jnp.sum((v - mean) ** 2, axis=0, keepdims=True) / (ne - 1)
            return var / (mean ** 2 + 1e-10)

        loss_ref[...] = (cv2(imp_ref[...]) + cv2(load_ref[...])) * 0.01


@jax.jit
def kernel(x, W1, W2):
    nb = N_TOK // BLOCK_T
    idxs_t, scores_t, loss = pl.pallas_call(
        _gate_body,
        grid=(nb,),
        in_specs=[
            pl.BlockSpec((BLOCK_T, D_MODEL), lambda i: (i, 0)),
            pl.BlockSpec((D_MODEL, N_EXPERTS), lambda i: (0, 0)),
            pl.BlockSpec((N_EXPERTS, N_EXPERTS), lambda i: (0, 0)),
        ],
        out_specs=[
            pl.BlockSpec((NUM_SELECTS, BLOCK_T), lambda i: (0, i)),
            pl.BlockSpec((NUM_SELECTS, BLOCK_T), lambda i: (0, i)),
            pl.BlockSpec((1, 1), lambda i: (0, 0)),
        ],
        out_shape=[
            jax.ShapeDtypeStruct((NUM_SELECTS, N_TOK), jnp.int32),
            jax.ShapeDtypeStruct((NUM_SELECTS, N_TOK), jnp.float32),
            jax.ShapeDtypeStruct((1, 1), jnp.float32),
        ],
        scratch_shapes=[
            pltpu.VMEM((N_EXPERTS, 1), jnp.float32),
            pltpu.VMEM((N_EXPERTS, 1), jnp.float32),
        ],
        compiler_params=pltpu.CompilerParams(
            dimension_semantics=("arbitrary",),
        ),
    )(x, W1.T, W2)
    return idxs_t.T, scores_t.T, jnp.reshape(loss, ())
